# baseline (speedup 1.0000x reference)
import functools

import numpy as np

import jax
import jax.numpy as jnp
from jax import lax
from jax.experimental import pallas as pl
from jax.experimental.pallas import tpu as pltpu
from jax.experimental.pallas import tpu_sc as plsc

_NUM_CLASSES = 1000
_PAD = 1024
_SS_BASE = 1001
_BATCH = 16384
_FEAT = 128
_LANES = 16
_NC = 2
_NS = 16
_ROWS_PER_TILE = _BATCH // (_NC * _NS)
_CHUNK = 128
_NCHUNK = _ROWS_PER_TILE // _CHUNK
_INIT_ROWS = _PAD // _NS

_ZEROS_NP = np.zeros((_PAD, _FEAT), np.float32)
_ONES_NP = np.ones((_CHUNK, _FEAT), np.float32)


def _sc_segment_sums(x, t2, zeros_hbm, ones_hbm):
    mesh = plsc.VectorSubcoreMesh(core_axis_name="c", subcore_axis_name="s")

    @functools.partial(
        pl.kernel,
        out_type=(
            jax.ShapeDtypeStruct((_NC, _PAD, _FEAT), jnp.float32),
            jax.ShapeDtypeStruct((_NC, _PAD, _FEAT), jnp.float32),
        ),
        mesh=mesh,
        scratch_types=[
            pltpu.VMEM_SHARED((_PAD, _FEAT), jnp.float32),
            pltpu.VMEM_SHARED((_PAD, _FEAT), jnp.float32),
            pltpu.VMEM_SHARED((_CHUNK, _FEAT), jnp.float32),
            pltpu.VMEM((_NCHUNK, _CHUNK), jnp.int32),
            pltpu.VMEM((_NCHUNK, _CHUNK, _FEAT), jnp.float32),
            pltpu.VMEM((_CHUNK, _FEAT), jnp.float32),
            pltpu.VMEM((1, _FEAT), jnp.float32),
            [pltpu.SemaphoreType.DMA] * _NCHUNK,
            pltpu.SemaphoreType.DMA,
            pltpu.SemaphoreType.DMA,
        ],
    )
    def k(x_hbm, t_hbm, z_hbm, ones_hbm_ref, out_s, out_c,
          acc, cnt, ones_stage, idx_v, rows_v, ones_v, ssbuf,
          sem_in, sem_sc, sem_init):
        core = lax.axis_index("c")
        sub = lax.axis_index("s")
        r0 = sub * _INIT_ROWS
        trow = core * (_NS * _NCHUNK) + sub * _NCHUNK
        row_base = core * (_BATCH // _NC) + sub * _ROWS_PER_TILE

        cps = [
            pltpu.async_copy(
                x_hbm.at[pl.ds(row_base + j * _CHUNK, _CHUNK)],
                rows_v.at[j], sem_in[j])
            for j in range(_NCHUNK)
        ]
        cp_idx = pltpu.async_copy(t_hbm.at[pl.ds(trow, _NCHUNK)], idx_v,
                                  sem_init)

        i16 = lax.iota(jnp.int32, _LANES)
        zero16 = (i16 * 0).astype(jnp.float32)
        for kk in range(_FEAT // _LANES):
            ssbuf[0, pl.ds(kk * _LANES, _LANES)] = zero16

        with jax.named_scope("init_wait"):
            @pl.when(sub == 0)
            def _():
                iz1 = pltpu.async_copy(z_hbm, acc, sem_init)
                iz2 = pltpu.async_copy(z_hbm, cnt, sem_init)
                io = pltpu.async_copy(ones_hbm_ref, ones_stage, sem_init)
                iz1.wait()
                iz2.wait()
                io.wait()

            cp_idx.wait()
            plsc.subcore_barrier()
            io2 = pltpu.async_copy(ones_stage, ones_v, sem_init)

        scs = []
        accv = zero16

        def ss_row(i, a, j):
            for kk in range(_FEAT // _LANES):
                v = rows_v[j, i, pl.ds(kk * _LANES, _LANES)]
                a = a + v * v
            return a

        with jax.named_scope("scatter"):
            for j in range(_NCHUNK):
                cps[j].wait()
                scs.append(pltpu.async_copy(rows_v.at[j], acc.at[idx_v.at[j]],
                                            sem_sc, add=True))
                if j == 0:
                    io2.wait()
                scs.append(pltpu.async_copy(ones_v, cnt.at[idx_v.at[j]],
                                            sem_sc, add=True))

        with jax.named_scope("ss_compute"):
            for j in range(_NCHUNK):
                accv = lax.fori_loop(0, _CHUNK,
                                     functools.partial(ss_row, j=j), accv)
            ssbuf[0, pl.ds(0, _LANES)] = accv
            scs.append(pltpu.async_copy(ssbuf,
                                        cnt.at[pl.ds(_SS_BASE + sub, 1)],
                                        sem_sc))

        with jax.named_scope("drain"):
            for d in scs:
                d.wait()
            plsc.subcore_barrier()

        with jax.named_scope("writeout"):
            o1 = pltpu.async_copy(acc.at[pl.ds(r0, _INIT_ROWS)],
                                  out_s.at[core, pl.ds(r0, _INIT_ROWS)],
                                  sem_init)
            o2 = pltpu.async_copy(cnt.at[pl.ds(r0, _INIT_ROWS)],
                                  out_c.at[core, pl.ds(r0, _INIT_ROWS)],
                                  sem_init)
            o1.wait()
            o2.wait()

    return k(x, t2, zeros_hbm, ones_hbm)


def _finalize(sums, cnts):
    def body(s_ref, c_ref, o_ref):
        s = s_ref[0] + s_ref[1]
        n = c_ref[0, :, 0:1] + c_ref[1, :, 0:1]
        sq = jnp.sum(s * s, axis=1, keepdims=True)
        row = lax.broadcasted_iota(jnp.int32, (_PAD, 1), 0)
        nz = (n > 0.0) & (row < _NUM_CLASSES)
        term = jnp.sum(jnp.where(nz, sq / jnp.where(nz, n, 1.0), 0.0))
        n_ids = jnp.sum(jnp.where(nz, 1.0, 0.0))
        total_ss = jnp.sum(c_ref[0, _SS_BASE:_SS_BASE + _NS, 0:_LANES]
                           + c_ref[1, _SS_BASE:_SS_BASE + _NS, 0:_LANES])
        loss = 0.5 * (total_ss - term)
        o_ref[...] = jnp.where(n_ids == float(_BATCH), 0.0, loss).reshape(1, 1)

    return pl.pallas_call(
        body,
        out_shape=jax.ShapeDtypeStruct((1, 1), jnp.float32),
    )(sums, cnts)


def kernel(inputs, targets):
    t2 = targets.reshape(_BATCH // _CHUNK, _CHUNK).astype(jnp.int32)
    zeros_hbm = jnp.asarray(_ZEROS_NP)
    ones_hbm = jnp.asarray(_ONES_NP)
    sums, cnts = _sc_segment_sums(inputs, t2, zeros_hbm, ones_hbm)
    out = _finalize(sums, cnts)
    return out[0, 0]


# speedup vs baseline: 1.0673x; 1.0673x over previous
import dataclasses
import functools

import numpy as np

import jax
import jax.numpy as jnp
from jax import lax
from jax.experimental import pallas as pl
from jax.experimental.pallas import tpu as pltpu
from jax.experimental.pallas import tpu_sc as plsc

_NUM_CLASSES = 1000
_PAD = 1024
_BATCH = 16384
_FEAT = 128
_LANES = 16
_NC = 2
_NS = 16
_ROWS_PER_TILE = _BATCH // (_NC * _NS)
_CHUNK = 128
_NCHUNK = _ROWS_PER_TILE // _CHUNK
_INIT_ROWS = _PAD // _NS
_HROWS = _PAD // _FEAT
_STATS = _HROWS + _NS

_ZEROS_NP = np.zeros((_PAD, _FEAT), np.float32)
_IDX8_NP = np.arange(_HROWS, dtype=np.int32)


def _sc_segment_sums(x, t2, zeros_hbm, idx8_hbm):
    mesh = plsc.VectorSubcoreMesh(core_axis_name="c", subcore_axis_name="s")
    cp = pltpu.CompilerParams()
    if "needs_layout_passes" in pltpu.CompilerParams.__dataclass_fields__:
        cp = dataclasses.replace(cp, needs_layout_passes=False)

    @functools.partial(
        pl.kernel,
        out_type=(
            jax.ShapeDtypeStruct((_NC, _PAD, _FEAT), jnp.float32),
            jax.ShapeDtypeStruct((_NC, _STATS, _FEAT), jnp.float32),
        ),
        mesh=mesh,
        compiler_params=cp,
        scratch_types=[
            pltpu.VMEM_SHARED((_PAD, _FEAT), jnp.float32),
            pltpu.VMEM_SHARED((_STATS, _FEAT), jnp.float32),
            pltpu.VMEM((_NCHUNK, _CHUNK), jnp.int32),
            pltpu.VMEM((_NCHUNK, _CHUNK, _FEAT), jnp.float32),
            pltpu.VMEM((_HROWS, _FEAT), jnp.float32),
            pltpu.VMEM((_HROWS,), jnp.int32),
            pltpu.VMEM((1, _FEAT), jnp.float32),
            [pltpu.SemaphoreType.DMA] * _NCHUNK,
            pltpu.SemaphoreType.DMA,
            pltpu.SemaphoreType.DMA,
        ],
    )
    def k(x_hbm, t_hbm, z_hbm, idx8_hbm_ref, out_s, out_stats,
          acc, stats, idx_v, rows_v, cnt_tile, idx8, ssbuf,
          sem_in, sem_sc, sem_init):
        core = lax.axis_index("c")
        sub = lax.axis_index("s")
        r0 = sub * _INIT_ROWS
        trow = core * (_NS * _NCHUNK) + sub * _NCHUNK
        row_base = core * (_BATCH // _NC) + sub * _ROWS_PER_TILE

        cps = [
            pltpu.async_copy(
                x_hbm.at[pl.ds(row_base + j * _CHUNK, _CHUNK)],
                rows_v.at[j], sem_in[j])
            for j in range(_NCHUNK)
        ]
        cp_idx = pltpu.async_copy(t_hbm.at[pl.ds(trow, _NCHUNK)], idx_v,
                                  sem_init)
        cp_idx8 = pltpu.async_copy(idx8_hbm_ref, idx8, sem_init)

        i16 = lax.iota(jnp.int32, _LANES)
        zero16 = (i16 * 0).astype(jnp.float32)
        one16 = zero16 + 1.0

        for r in range(_HROWS):
            for kk in range(_FEAT // _LANES):
                cnt_tile[r, pl.ds(kk * _LANES, _LANES)] = zero16

        with jax.named_scope("init_wait"):
            @pl.when(sub == 0)
            def _():
                iz1 = pltpu.async_copy(z_hbm, acc, sem_init)
                iz2 = pltpu.async_copy(z_hbm.at[pl.ds(0, _STATS)], stats,
                                       sem_init)
                iz1.wait()
                iz2.wait()

            cp_idx.wait()
            cp_idx8.wait()

        with jax.named_scope("histogram"):
            def hist_step(i, carry):
                j = i // (_CHUNK // _LANES)
                kk = lax.rem(i, _CHUNK // _LANES)
                t16 = idx_v[j, pl.ds(kk * _LANES, _LANES)]
                rows = lax.shift_right_logical(t16, 7)
                cols = lax.bitwise_and(t16, _FEAT - 1)
                plsc.addupdate_scatter(cnt_tile, [rows, cols], one16)
                return carry

            lax.fori_loop(0, _ROWS_PER_TILE // _LANES, hist_step, 0)
            plsc.subcore_barrier()
            cm = pltpu.async_copy(cnt_tile, stats.at[idx8], sem_sc, add=True)

        scs = [cm]
        accv = zero16

        def ss_row(i, a, j):
            for kk in range(_FEAT // _LANES):
                v = rows_v[j, i, pl.ds(kk * _LANES, _LANES)]
                a = a + v * v
            return a

        with jax.named_scope("scatter"):
            for j in range(_NCHUNK):
                cps[j].wait()
                scs.append(pltpu.async_copy(rows_v.at[j], acc.at[idx_v.at[j]],
                                            sem_sc, add=True))

        with jax.named_scope("ss_compute"):
            for j in range(_NCHUNK):
                accv = lax.fori_loop(0, _CHUNK,
                                     functools.partial(ss_row, j=j), accv)
            ssbuf[0, pl.ds(0, _LANES)] = accv
            scs.append(pltpu.async_copy(ssbuf,
                                        stats.at[pl.ds(_HROWS + sub, 1)],
                                        sem_sc))

        with jax.named_scope("drain"):
            for d in scs:
                d.wait()
            plsc.subcore_barrier()

        with jax.named_scope("writeout"):
            o1 = pltpu.async_copy(acc.at[pl.ds(r0, _INIT_ROWS)],
                                  out_s.at[core, pl.ds(r0, _INIT_ROWS)],
                                  sem_init)

            @pl.when(sub == 0)
            def _():
                pltpu.async_copy(stats, out_stats.at[core], sem_init).wait()

            o1.wait()

    return k(x, t2, zeros_hbm, idx8_hbm)


def _finalize(sums4, stats):
    def body(s_ref, c_ref, o_ref):
        counts8 = c_ref[0, 0:_HROWS, :] + c_ref[1, 0:_HROWS, :]
        counts_t = jnp.transpose(counts8)
        sub_iota = lax.broadcasted_iota(jnp.int32, (_FEAT, 1), 0)
        term = jnp.float32(0.0)
        n_ids = jnp.float32(0.0)
        for r in range(_HROWS):
            block = s_ref[0, r] + s_ref[1, r]
            sq = jnp.sum(block * block, axis=1, keepdims=True)
            n = counts_t[:, r:r + 1]
            nz = (n > 0.0) & (r * _FEAT + sub_iota < _NUM_CLASSES)
            term += jnp.sum(jnp.where(nz, sq / jnp.where(nz, n, 1.0), 0.0))
            n_ids += jnp.sum(jnp.where(nz, 1.0, 0.0))
        total_ss = jnp.sum(c_ref[0, _HROWS:_STATS, 0:_LANES]
                           + c_ref[1, _HROWS:_STATS, 0:_LANES])
        loss = 0.5 * (total_ss - term)
        o_ref[...] = jnp.where(n_ids == float(_BATCH), 0.0, loss).reshape(1, 1)

    return pl.pallas_call(
        body,
        out_shape=jax.ShapeDtypeStruct((1, 1), jnp.float32),
    )(sums4, stats)


def kernel(inputs, targets):
    t2 = targets.reshape(_BATCH // _CHUNK, _CHUNK).astype(jnp.int32)
    zeros_hbm = jnp.asarray(_ZEROS_NP)
    idx8_hbm = jnp.asarray(_IDX8_NP)
    sums, stats = _sc_segment_sums(inputs, t2, zeros_hbm, idx8_hbm)
    out = _finalize(sums.reshape(_NC, _HROWS, _FEAT, _FEAT), stats)
    return out[0, 0]


# speedup vs baseline: 1.0977x; 1.0285x over previous
import dataclasses
import functools

import numpy as np

import jax
import jax.numpy as jnp
from jax import lax
from jax.experimental import pallas as pl
from jax.experimental.pallas import tpu as pltpu
from jax.experimental.pallas import tpu_sc as plsc

_NUM_CLASSES = 1000
_PAD = 1024
_BATCH = 16384
_FEAT = 128
_LANES = 16
_NC = 2
_NS = 16
_ROWS_PER_TILE = _BATCH // (_NC * _NS)
_CHUNK = 128
_NCHUNK = _ROWS_PER_TILE // _CHUNK
_INIT_ROWS = _PAD // _NS
_HROWS = _PAD // _FEAT
_STATS = _HROWS

_ZEROS_NP = np.zeros((_PAD, _FEAT), np.float32)
_IDX8_NP = np.arange(_HROWS, dtype=np.int32)


def _sc_segment_sums(x, t2, zeros_hbm, idx8_hbm):
    mesh = plsc.VectorSubcoreMesh(core_axis_name="c", subcore_axis_name="s")
    cp = pltpu.CompilerParams()
    if "needs_layout_passes" in pltpu.CompilerParams.__dataclass_fields__:
        cp = dataclasses.replace(cp, needs_layout_passes=False)

    @functools.partial(
        pl.kernel,
        out_type=(
            jax.ShapeDtypeStruct((_NC, _PAD, _FEAT), jnp.float32),
            jax.ShapeDtypeStruct((_NC, _STATS, _FEAT), jnp.float32),
        ),
        mesh=mesh,
        compiler_params=cp,
        scratch_types=[
            pltpu.VMEM_SHARED((_PAD, _FEAT), jnp.float32),
            pltpu.VMEM_SHARED((_STATS, _FEAT), jnp.float32),
            pltpu.VMEM((_NCHUNK, _CHUNK), jnp.int32),
            pltpu.VMEM((_NCHUNK, _CHUNK, _FEAT), jnp.float32),
            pltpu.VMEM((_HROWS, _FEAT), jnp.float32),
            pltpu.VMEM((_HROWS,), jnp.int32),
            [pltpu.SemaphoreType.DMA] * _NCHUNK,
            pltpu.SemaphoreType.DMA,
            pltpu.SemaphoreType.DMA,
        ],
    )
    def k(x_hbm, t_hbm, z_hbm, idx8_hbm_ref, out_s, out_stats,
          acc, stats, idx_v, rows_v, cnt_tile, idx8,
          sem_in, sem_sc, sem_init):
        core = lax.axis_index("c")
        sub = lax.axis_index("s")
        r0 = sub * _INIT_ROWS
        trow = core * (_NS * _NCHUNK) + sub * _NCHUNK
        row_base = core * (_BATCH // _NC) + sub * _ROWS_PER_TILE

        cps = [
            pltpu.async_copy(
                x_hbm.at[pl.ds(row_base + j * _CHUNK, _CHUNK)],
                rows_v.at[j], sem_in[j])
            for j in range(_NCHUNK)
        ]
        cp_idx = pltpu.async_copy(t_hbm.at[pl.ds(trow, _NCHUNK)], idx_v,
                                  sem_init)
        cp_idx8 = pltpu.async_copy(idx8_hbm_ref, idx8, sem_init)

        i16 = lax.iota(jnp.int32, _LANES)
        zero16 = (i16 * 0).astype(jnp.float32)
        one16 = zero16 + 1.0

        for r in range(_HROWS):
            for kk in range(_FEAT // _LANES):
                cnt_tile[r, pl.ds(kk * _LANES, _LANES)] = zero16

        with jax.named_scope("init_wait"):
            @pl.when(sub == 0)
            def _():
                iz1 = pltpu.async_copy(z_hbm, acc, sem_init)
                iz2 = pltpu.async_copy(z_hbm.at[pl.ds(0, _STATS)], stats,
                                       sem_init)
                iz1.wait()
                iz2.wait()

            cp_idx.wait()
            cp_idx8.wait()

        with jax.named_scope("histogram"):
            def hist_step(i, carry):
                j = i // (_CHUNK // _LANES)
                kk = lax.rem(i, _CHUNK // _LANES)
                t16 = idx_v[j, pl.ds(kk * _LANES, _LANES)]
                rows = lax.shift_right_logical(t16, 7)
                cols = lax.bitwise_and(t16, _FEAT - 1)
                plsc.addupdate_scatter(cnt_tile, [rows, cols], one16)
                return carry

            lax.fori_loop(0, _ROWS_PER_TILE // _LANES, hist_step, 0)
            plsc.subcore_barrier()
            cm = pltpu.async_copy(cnt_tile, stats.at[idx8], sem_sc, add=True)

        scs = [cm]

        with jax.named_scope("scatter"):
            for j in range(_NCHUNK):
                cps[j].wait()
                scs.append(pltpu.async_copy(rows_v.at[j], acc.at[idx_v.at[j]],
                                            sem_sc, add=True))

        with jax.named_scope("drain"):
            for d in scs:
                d.wait()
            plsc.subcore_barrier()

        with jax.named_scope("writeout"):
            o1 = pltpu.async_copy(acc.at[pl.ds(r0, _INIT_ROWS)],
                                  out_s.at[core, pl.ds(r0, _INIT_ROWS)],
                                  sem_init)

            @pl.when(sub == 0)
            def _():
                pltpu.async_copy(stats, out_stats.at[core], sem_init).wait()

            o1.wait()

    return k(x, t2, zeros_hbm, idx8_hbm)


def _sumsq(x):
    def body(x_ref, o_ref):
        @pl.when(pl.program_id(0) == 0)
        def _():
            o_ref[...] = jnp.zeros_like(o_ref)
        xb = x_ref[...]
        o_ref[...] += jnp.sum(xb * xb, axis=0, keepdims=True)

    return pl.pallas_call(
        body,
        grid=(_BATCH // 2048,),
        in_specs=[pl.BlockSpec((2048, _FEAT), lambda i: (i, 0))],
        out_specs=pl.BlockSpec((1, _FEAT), lambda i: (0, 0)),
        out_shape=jax.ShapeDtypeStruct((1, _FEAT), jnp.float32),
    )(x)


def _finalize(sums4, stats, ss):
    def body(s_ref, c_ref, ss_ref, o_ref):
        counts8 = c_ref[0, 0:_HROWS, :] + c_ref[1, 0:_HROWS, :]
        counts_t = jnp.transpose(counts8)
        sub_iota = lax.broadcasted_iota(jnp.int32, (_FEAT, 1), 0)
        term = jnp.float32(0.0)
        n_ids = jnp.float32(0.0)
        for r in range(_HROWS):
            block = s_ref[0, r] + s_ref[1, r]
            sq = jnp.sum(block * block, axis=1, keepdims=True)
            n = counts_t[:, r:r + 1]
            nz = (n > 0.0) & (r * _FEAT + sub_iota < _NUM_CLASSES)
            term += jnp.sum(jnp.where(nz, sq / jnp.where(nz, n, 1.0), 0.0))
            n_ids += jnp.sum(jnp.where(nz, 1.0, 0.0))
        total_ss = jnp.sum(ss_ref[...])
        loss = 0.5 * (total_ss - term)
        o_ref[...] = jnp.where(n_ids == float(_BATCH), 0.0, loss).reshape(1, 1)

    return pl.pallas_call(
        body,
        out_shape=jax.ShapeDtypeStruct((1, 1), jnp.float32),
    )(sums4, stats, ss)


def kernel(inputs, targets):
    t2 = targets.reshape(_BATCH // _CHUNK, _CHUNK).astype(jnp.int32)
    zeros_hbm = jnp.asarray(_ZEROS_NP)
    idx8_hbm = jnp.asarray(_IDX8_NP)
    sums, stats = _sc_segment_sums(inputs, t2, zeros_hbm, idx8_hbm)
    ss = _sumsq(inputs)
    out = _finalize(sums.reshape(_NC, _HROWS, _FEAT, _FEAT), stats, ss)
    return out[0, 0]


# speedup vs baseline: 1.0980x; 1.0003x over previous
import dataclasses
import functools

import numpy as np

import jax
import jax.numpy as jnp
from jax import lax
from jax.experimental import pallas as pl
from jax.experimental.pallas import tpu as pltpu
from jax.experimental.pallas import tpu_sc as plsc

_NUM_CLASSES = 1000
_PAD = 1024
_BATCH = 16384
_FEAT = 128
_LANES = 16
_NC = 2
_NS = 16
_ROWS_PER_TILE = _BATCH // (_NC * _NS)
_CHUNK = 128
_NCHUNK = _ROWS_PER_TILE // _CHUNK
_INIT_ROWS = _PAD // _NS
_HROWS = _PAD // _FEAT
_STATS = _HROWS

_ZEROS_NP = np.zeros((_PAD, _FEAT), np.float32)
_IDX8_NP = np.arange(_HROWS, dtype=np.int32)


def _sc_segment_sums(x, t2, zeros_hbm, idx8_hbm):
    mesh = plsc.VectorSubcoreMesh(core_axis_name="c", subcore_axis_name="s")
    cp = pltpu.CompilerParams()
    if "needs_layout_passes" in pltpu.CompilerParams.__dataclass_fields__:
        cp = dataclasses.replace(cp, needs_layout_passes=False)

    @functools.partial(
        pl.kernel,
        out_type=(
            jax.ShapeDtypeStruct((_NC, _PAD, _FEAT), jnp.float32),
            jax.ShapeDtypeStruct((_NC, _STATS, _FEAT), jnp.float32),
        ),
        mesh=mesh,
        compiler_params=cp,
        scratch_types=[
            pltpu.VMEM_SHARED((_PAD, _FEAT), jnp.float32),
            pltpu.VMEM_SHARED((_STATS, _FEAT), jnp.float32),
            pltpu.VMEM((_NCHUNK, _CHUNK), jnp.int32),
            pltpu.VMEM((_NCHUNK, _CHUNK, _FEAT), jnp.float32),
            pltpu.VMEM((_HROWS, _FEAT), jnp.float32),
            pltpu.VMEM((_HROWS,), jnp.int32),
            [pltpu.SemaphoreType.DMA] * _NCHUNK,
            pltpu.SemaphoreType.DMA,
            pltpu.SemaphoreType.DMA,
        ],
    )
    def k(x_hbm, t_hbm, z_hbm, idx8_hbm_ref, out_s, out_stats,
          acc, stats, idx_v, rows_v, cnt_tile, idx8,
          sem_in, sem_sc, sem_init):
        core = lax.axis_index("c")
        sub = lax.axis_index("s")
        r0 = sub * _INIT_ROWS
        trow = core * (_NS * _NCHUNK) + sub * _NCHUNK
        row_base = core * (_BATCH // _NC) + sub * _ROWS_PER_TILE

        cps = [
            pltpu.async_copy(
                x_hbm.at[pl.ds(row_base + j * _CHUNK, _CHUNK)],
                rows_v.at[j], sem_in[j])
            for j in range(_NCHUNK)
        ]
        cp_idx = pltpu.async_copy(t_hbm.at[pl.ds(trow, _NCHUNK)], idx_v,
                                  sem_init)
        cp_idx8 = pltpu.async_copy(idx8_hbm_ref, idx8, sem_init)

        i16 = lax.iota(jnp.int32, _LANES)
        zero16 = (i16 * 0).astype(jnp.float32)
        one16 = zero16 + 1.0

        for r in range(_HROWS):
            for kk in range(_FEAT // _LANES):
                cnt_tile[r, pl.ds(kk * _LANES, _LANES)] = zero16

        with jax.named_scope("init_wait"):
            iz1 = pltpu.async_copy(z_hbm.at[pl.ds(r0, _INIT_ROWS)],
                                   acc.at[pl.ds(r0, _INIT_ROWS)], sem_init)

            @pl.when(sub == 0)
            def _():
                pltpu.async_copy(z_hbm.at[pl.ds(0, _STATS)], stats,
                                 sem_init).wait()

            iz1.wait()
            cp_idx.wait()
            cp_idx8.wait()

        with jax.named_scope("histogram"):
            def hist_step(i, carry):
                j = i // (_CHUNK // _LANES)
                kk = lax.rem(i, _CHUNK // _LANES)
                t16 = idx_v[j, pl.ds(kk * _LANES, _LANES)]
                rows = lax.shift_right_logical(t16, 7)
                cols = lax.bitwise_and(t16, _FEAT - 1)
                plsc.addupdate_scatter(cnt_tile, [rows, cols], one16)
                return carry

            lax.fori_loop(0, _ROWS_PER_TILE // _LANES, hist_step, 0)
            plsc.subcore_barrier()
            cm = pltpu.async_copy(cnt_tile, stats.at[idx8], sem_sc, add=True)

        scs = [cm]

        with jax.named_scope("scatter"):
            for j in range(_NCHUNK):
                cps[j].wait()
                scs.append(pltpu.async_copy(rows_v.at[j], acc.at[idx_v.at[j]],
                                            sem_sc, add=True))

        with jax.named_scope("drain"):
            for d in scs:
                d.wait()
            plsc.subcore_barrier()

        with jax.named_scope("writeout"):
            o1 = pltpu.async_copy(acc.at[pl.ds(r0, _INIT_ROWS)],
                                  out_s.at[core, pl.ds(r0, _INIT_ROWS)],
                                  sem_init)

            @pl.when(sub == 0)
            def _():
                pltpu.async_copy(stats, out_stats.at[core], sem_init).wait()

            o1.wait()

    return k(x, t2, zeros_hbm, idx8_hbm)


def _sumsq(x):
    def body(x_ref, o_ref):
        @pl.when(pl.program_id(0) == 0)
        def _():
            o_ref[...] = jnp.zeros_like(o_ref)
        xb = x_ref[...]
        o_ref[...] += jnp.sum(xb * xb, axis=0, keepdims=True)

    return pl.pallas_call(
        body,
        grid=(_BATCH // 2048,),
        in_specs=[pl.BlockSpec((2048, _FEAT), lambda i: (i, 0))],
        out_specs=pl.BlockSpec((1, _FEAT), lambda i: (0, 0)),
        out_shape=jax.ShapeDtypeStruct((1, _FEAT), jnp.float32),
    )(x)


def _finalize(sums4, stats, ss):
    def body(s_ref, c_ref, ss_ref, o_ref):
        counts8 = c_ref[0, 0:_HROWS, :] + c_ref[1, 0:_HROWS, :]
        counts_t = jnp.transpose(counts8)
        sub_iota = lax.broadcasted_iota(jnp.int32, (_FEAT, 1), 0)
        term = jnp.float32(0.0)
        n_ids = jnp.float32(0.0)
        for r in range(_HROWS):
            block = s_ref[0, r] + s_ref[1, r]
            sq = jnp.sum(block * block, axis=1, keepdims=True)
            n = counts_t[:, r:r + 1]
            nz = (n > 0.0) & (r * _FEAT + sub_iota < _NUM_CLASSES)
            term += jnp.sum(jnp.where(nz, sq / jnp.where(nz, n, 1.0), 0.0))
            n_ids += jnp.sum(jnp.where(nz, 1.0, 0.0))
        total_ss = jnp.sum(ss_ref[...])
        loss = 0.5 * (total_ss - term)
        o_ref[...] = jnp.where(n_ids == float(_BATCH), 0.0, loss).reshape(1, 1)

    return pl.pallas_call(
        body,
        out_shape=jax.ShapeDtypeStruct((1, 1), jnp.float32),
    )(sums4, stats, ss)


def kernel(inputs, targets):
    t2 = targets.reshape(_BATCH // _CHUNK, _CHUNK).astype(jnp.int32)
    zeros_hbm = jnp.asarray(_ZEROS_NP)
    idx8_hbm = jnp.asarray(_IDX8_NP)
    sums, stats = _sc_segment_sums(inputs, t2, zeros_hbm, idx8_hbm)
    ss = _sumsq(inputs)
    out = _finalize(sums.reshape(_NC, _HROWS, _FEAT, _FEAT), stats, ss)
    return out[0, 0]
